# baseline (speedup 1.0000x reference)
import functools

import jax
import jax.numpy as jnp
from jax import lax
from jax.experimental import pallas as pl
from jax.experimental.pallas import tpu as pltpu
from jax.experimental.pallas import tpu_sc as plsc

_VOCAB = 1000000
_MAXLEN = 200
_EMBED = 64
_BATCH = 1024

_INFO = plsc.get_sparse_core_info()
_NC, _NS, _L = _INFO.num_cores, _INFO.num_subcores, _INFO.num_lanes
_NW = _NC * _NS
_BB = _BATCH // 128
_UNITS = _MAXLEN * _BB
_UPW = _UNITS // _NW
_GPB = 128 // _L


def _body(xt_hbm, tok_hbm, pos_hbm, out_hbm,
          pos_v, ids_v, idx_v, par_v, bufs, slabs, gsem, osem):
    wid = lax.axis_index("s") * _NC + lax.axis_index("c")
    ubase = wid * _UPW

    pltpu.sync_copy(pos_hbm, pos_v)

    def stage_ids_and_gather(u, slot):
        s = u >> 3
        bb = u & 7
        pltpu.sync_copy(xt_hbm.at[pl.ds(s * _BATCH + bb * 128, 128)],
                        ids_v.at[slot])
        for g in range(_GPB):
            sl = pl.ds(g * _L, _L)
            ids = ids_v[slot, sl]
            idx_v[slot, sl] = lax.shift_right_logical(ids, 1)
            par_v[slot, sl] = lax.shift_left(jnp.bitwise_and(ids, 1), 6)
        g_copy(u, slot).start()

    def g_copy(u, slot):
        return pltpu.make_async_copy(
            tok_hbm.at[idx_v.at[slot]], bufs[slot], gsem.at[slot])

    def o_copy(u, slot):
        s = u >> 3
        bb = u & 7
        return pltpu.make_async_copy(
            slabs[slot], out_hbm.at[s, :, pl.ds(bb * 128, 128)],
            osem.at[slot])

    def transpose_add(u, slot):
        s = u >> 3
        buf = bufs[slot]
        slab = slabs[slot]

        def erow(e, carry):
            zs = jnp.zeros((_L,), jnp.int32)
            p = plsc.load_gather(pos_v, [zs + s, zs + e])
            for g in range(_GPB):
                sl = pl.ds(g * _L, _L)
                row_idx = jax.lax.iota(jnp.int32, _L) + (g * _L)
                col_idx = par_v[slot, sl] + e
                vals = plsc.load_gather(buf, [row_idx, col_idx])
                slab[e, sl] = vals + p
            return carry

        lax.fori_loop(0, _EMBED, erow, 0)

    stage_ids_and_gather(ubase, 0)

    def pair_body(pr, carry):
        for slot in range(2):
            ul = pr * 2 + slot
            u = ubase + ul
            nslot = 1 - slot

            @pl.when(ul + 1 < _UPW)
            def _():
                stage_ids_and_gather(u + 1, nslot)

            g_copy(u, slot).wait()

            @pl.when(ul >= 2)
            def _():
                o_copy(u - 2, slot).wait()

            transpose_add(u, slot)
            o_copy(u, slot).start()
        return carry

    lax.fori_loop(0, _UPW // 2, pair_body, 0)

    o_copy(ubase + _UPW - 2, 0).wait()
    o_copy(ubase + _UPW - 1, 1).wait()


@jax.jit
def _run(xt_flat, tok_r, pos_t):
    mesh = plsc.VectorSubcoreMesh(core_axis_name="c", subcore_axis_name="s")

    def wrapped(xt_hbm, tok_hbm, pos_hbm, out_hbm, pos_v, ids_v, idx_v,
                par_v, b0, b1, s0, s1, gsem, osem):
        _body(xt_hbm, tok_hbm, pos_hbm, out_hbm, pos_v, ids_v, idx_v,
              par_v, (b0, b1), (s0, s1), gsem, osem)

    k = functools.partial(
        pl.kernel,
        mesh=mesh,
        out_type=jax.ShapeDtypeStruct((_MAXLEN, _EMBED, _BATCH), jnp.float32),
        scratch_types=[
            pltpu.VMEM((_MAXLEN, _EMBED), jnp.float32),
            pltpu.VMEM((2, 128), jnp.int32),
            pltpu.VMEM((2, 128), jnp.int32),
            pltpu.VMEM((2, 128), jnp.int32),
            pltpu.VMEM((128, 128), jnp.float32),
            pltpu.VMEM((128, 128), jnp.float32),
            pltpu.VMEM((_EMBED, 128), jnp.float32),
            pltpu.VMEM((_EMBED, 128), jnp.float32),
            pltpu.SemaphoreType.DMA((2,)),
            pltpu.SemaphoreType.DMA((2,)),
        ],
        compiler_params=pltpu.CompilerParams(needs_layout_passes=False),
    )(wrapped)
    return k(xt_flat, tok_r, pos_t)


def kernel(x, token_emb, pos_emb):
    xt_flat = x.T.reshape(-1).astype(jnp.int32)
    tok_r = token_emb.reshape(_VOCAB // 2, 2 * _EMBED)
    out6 = _run(xt_flat, tok_r, pos_emb)
    return out6.transpose(2, 0, 1)


# speedup vs baseline: 1.0191x; 1.0191x over previous
import functools

import jax
import jax.numpy as jnp
from jax import lax
from jax.experimental import pallas as pl
from jax.experimental.pallas import tpu as pltpu
from jax.experimental.pallas import tpu_sc as plsc

_VOCAB = 1000000
_MAXLEN = 200
_EMBED = 64
_BATCH = 1024

_INFO = plsc.get_sparse_core_info()
_NC, _NS, _L = _INFO.num_cores, _INFO.num_subcores, _INFO.num_lanes
_NW = _NC * _NS
_BB = _BATCH // 128
_UNITS = _MAXLEN * _BB
_UPW = _UNITS // _NW
_IPW = _UPW * 128
_GPB = 128 // _L
_NB = 4


def _body(xt_hbm, tok_hbm, pos_hbm, out_hbm,
          pos_v, ids_v, idx_v, par_v, bufs, slabs, gsem, osem):
    wid = lax.axis_index("s") * _NC + lax.axis_index("c")
    ubase = wid * _UPW

    pltpu.sync_copy(pos_hbm, pos_v)
    pltpu.sync_copy(xt_hbm.at[pl.ds(ubase * 128, _IPW)], ids_v)

    def prep(i, carry):
        sl = pl.ds(i * _L, _L)
        ids = ids_v[sl]
        idx_v[sl] = lax.shift_right_logical(ids, 1)
        par_v[sl] = lax.shift_left(jnp.bitwise_and(ids, 1), 6)
        return carry

    lax.fori_loop(0, _IPW // _L, prep, 0, unroll=4)

    def g_copy(ul, slot):
        return pltpu.make_async_copy(
            tok_hbm.at[idx_v.at[pl.ds(ul * 128, 128)]],
            bufs[slot], gsem.at[slot])

    def o_copy(ul, sslot):
        u = ubase + ul
        s = u >> 3
        bb = u & 7
        return pltpu.make_async_copy(
            slabs[sslot], out_hbm.at[s, :, pl.ds(bb * 128, 128)],
            osem.at[sslot])

    def transpose_add(ul, slot, sslot):
        u = ubase + ul
        s = u >> 3
        buf = bufs[slot]
        slab = slabs[sslot]
        zs = jnp.zeros((_L,), jnp.int32)
        pbase = ul * 128

        def erow(e, carry):
            p = plsc.load_gather(pos_v, [zs + s, zs + e])
            for g in range(_GPB):
                sl = pl.ds(g * _L, _L)
                row_idx = jax.lax.iota(jnp.int32, _L) + (g * _L)
                col_idx = par_v[pl.ds(pbase + g * _L, _L)] + e
                vals = plsc.load_gather(buf, [row_idx, col_idx])
                slab[e, sl] = vals + p
            return carry

        lax.fori_loop(0, _EMBED, erow, 0, unroll=2)

    for j in range(_NB):
        g_copy(j, j).start()

    def quad_body(i, carry):
        for j in range(_NB):
            ul = i * _NB + j
            g_copy(ul, j).wait()

            @pl.when(ul >= 2)
            def _():
                o_copy(ul - 2, j % 2).wait()

            transpose_add(ul, j, j % 2)
            o_copy(ul, j % 2).start()

            @pl.when(ul + _NB < _UPW)
            def _():
                g_copy(ul + _NB, j).start()
        return carry

    lax.fori_loop(0, _UPW // 2 // _NB * 2, quad_body, 0)

    for j in range(2):
        ul = (_UPW // 2 // _NB * 2) * _NB + j
        g_copy(ul, j).wait()
        o_copy(ul - 2, j % 2).wait()
        transpose_add(ul, j, j % 2)
        o_copy(ul, j % 2).start()

    o_copy(_UPW - 2, 0).wait()
    o_copy(_UPW - 1, 1).wait()


@jax.jit
def _run(xt_flat, tok_r, pos_e):
    mesh = plsc.VectorSubcoreMesh(core_axis_name="c", subcore_axis_name="s")

    def wrapped(xt_hbm, tok_hbm, pos_hbm, out_hbm, pos_v, ids_v, idx_v,
                par_v, b0, b1, b2, b3, s0, s1, gsem, osem):
        _body(xt_hbm, tok_hbm, pos_hbm, out_hbm, pos_v, ids_v, idx_v,
              par_v, (b0, b1, b2, b3), (s0, s1), gsem, osem)

    k = functools.partial(
        pl.kernel,
        mesh=mesh,
        out_type=jax.ShapeDtypeStruct((_MAXLEN, _EMBED, _BATCH), jnp.float32),
        scratch_types=[
            pltpu.VMEM((_MAXLEN, _EMBED), jnp.float32),
            pltpu.VMEM((_IPW,), jnp.int32),
            pltpu.VMEM((_IPW,), jnp.int32),
            pltpu.VMEM((_IPW,), jnp.int32),
            pltpu.VMEM((128, 128), jnp.float32),
            pltpu.VMEM((128, 128), jnp.float32),
            pltpu.VMEM((128, 128), jnp.float32),
            pltpu.VMEM((128, 128), jnp.float32),
            pltpu.VMEM((_EMBED, 128), jnp.float32),
            pltpu.VMEM((_EMBED, 128), jnp.float32),
            pltpu.SemaphoreType.DMA((_NB,)),
            pltpu.SemaphoreType.DMA((2,)),
        ],
        compiler_params=pltpu.CompilerParams(needs_layout_passes=False),
    )(wrapped)
    return k(xt_flat, tok_r, pos_e)


def kernel(x, token_emb, pos_emb):
    xt_flat = x.T.reshape(-1).astype(jnp.int32)
    tok_r = token_emb.reshape(_VOCAB // 2, 2 * _EMBED)
    out6 = _run(xt_flat, tok_r, pos_emb)
    return out6.transpose(2, 0, 1)


# speedup vs baseline: 1.4185x; 1.3919x over previous
import functools

import jax
import jax.numpy as jnp
from jax import lax
from jax.experimental import pallas as pl
from jax.experimental.pallas import tpu as pltpu
from jax.experimental.pallas import tpu_sc as plsc

_VOCAB = 1000000
_MAXLEN = 200
_EMBED = 64
_BATCH = 1024

_INFO = plsc.get_sparse_core_info()
_NC, _NS, _L = _INFO.num_cores, _INFO.num_subcores, _INFO.num_lanes
_NW = _NC * _NS
_BB = _BATCH // 128
_UNITS = _MAXLEN * _BB
_UPW = _UNITS // _NW
_IPW = _UPW * 128
_GPB = 128 // _L
_NB = 4


def _body(xt_hbm, tok_hbm, pos_hbm, out_hbm,
          pos_v, ids_v, idx_v, par_v, bufs, slabs, gsem, osem):
    wid = lax.axis_index("s") * _NC + lax.axis_index("c")
    ubase = wid * _UPW

    pltpu.sync_copy(pos_hbm, pos_v)
    pltpu.sync_copy(xt_hbm.at[pl.ds(ubase * 128, _IPW)], ids_v)

    @plsc.parallel_loop(0, _IPW // _L, unroll=4)
    def _prep(i):
        sl = pl.ds(i * _L, _L)
        ids = ids_v[sl]
        idx_v[sl] = lax.shift_right_logical(ids, 1)
        par_v[sl] = lax.shift_left(jnp.bitwise_and(ids, 1), 6)

    def g_copy(ul, slot):
        return pltpu.make_async_copy(
            tok_hbm.at[idx_v.at[pl.ds(ul * 128, 128)]],
            bufs[slot], gsem.at[slot])

    def o_copy(ul, sslot):
        u = ubase + ul
        s = u >> 3
        bb = u & 7
        return pltpu.make_async_copy(
            slabs[sslot], out_hbm.at[s, :, pl.ds(bb * 128, 128)],
            osem.at[sslot])

    def transpose_add(ul, slot, sslot):
        u = ubase + ul
        s = u >> 3
        buf = bufs[slot]
        slab = slabs[sslot]
        zs = jnp.zeros((_L,), jnp.int32)
        pbase = ul * 128

        @plsc.parallel_loop(0, _EMBED, unroll=2)
        def _erow(e):
            p = plsc.load_gather(pos_v, [zs + s, zs + e])
            for g in range(_GPB):
                sl = pl.ds(g * _L, _L)
                row_idx = jax.lax.iota(jnp.int32, _L) + (g * _L)
                col_idx = par_v[pl.ds(pbase + g * _L, _L)] + e
                vals = plsc.load_gather(buf, [row_idx, col_idx])
                slab[e, sl] = vals + p

    for j in range(_NB):
        g_copy(j, j).start()

    def quad_body(i, carry):
        for j in range(_NB):
            ul = i * _NB + j
            g_copy(ul, j).wait()

            @pl.when(ul >= 2)
            def _():
                o_copy(ul - 2, j % 2).wait()

            transpose_add(ul, j, j % 2)
            o_copy(ul, j % 2).start()

            @pl.when(ul + _NB < _UPW)
            def _():
                g_copy(ul + _NB, j).start()
        return carry

    lax.fori_loop(0, _UPW // 2 // _NB * 2, quad_body, 0)

    for j in range(2):
        ul = (_UPW // 2 // _NB * 2) * _NB + j
        g_copy(ul, j).wait()
        o_copy(ul - 2, j % 2).wait()
        transpose_add(ul, j, j % 2)
        o_copy(ul, j % 2).start()

    o_copy(_UPW - 2, 0).wait()
    o_copy(_UPW - 1, 1).wait()


@jax.jit
def _run(xt_flat, tok_r, pos_e):
    mesh = plsc.VectorSubcoreMesh(core_axis_name="c", subcore_axis_name="s")

    def wrapped(xt_hbm, tok_hbm, pos_hbm, out_hbm, pos_v, ids_v, idx_v,
                par_v, b0, b1, b2, b3, s0, s1, gsem, osem):
        _body(xt_hbm, tok_hbm, pos_hbm, out_hbm, pos_v, ids_v, idx_v,
              par_v, (b0, b1, b2, b3), (s0, s1), gsem, osem)

    k = functools.partial(
        pl.kernel,
        mesh=mesh,
        out_type=jax.ShapeDtypeStruct((_MAXLEN, _EMBED, _BATCH), jnp.float32),
        scratch_types=[
            pltpu.VMEM((_MAXLEN, _EMBED), jnp.float32),
            pltpu.VMEM((_IPW,), jnp.int32),
            pltpu.VMEM((_IPW,), jnp.int32),
            pltpu.VMEM((_IPW,), jnp.int32),
            pltpu.VMEM((128, 128), jnp.float32),
            pltpu.VMEM((128, 128), jnp.float32),
            pltpu.VMEM((128, 128), jnp.float32),
            pltpu.VMEM((128, 128), jnp.float32),
            pltpu.VMEM((_EMBED, 128), jnp.float32),
            pltpu.VMEM((_EMBED, 128), jnp.float32),
            pltpu.SemaphoreType.DMA((_NB,)),
            pltpu.SemaphoreType.DMA((2,)),
        ],
        compiler_params=pltpu.CompilerParams(needs_layout_passes=False),
    )(wrapped)
    return k(xt_flat, tok_r, pos_e)


def kernel(x, token_emb, pos_emb):
    xt_flat = x.T.reshape(-1).astype(jnp.int32)
    tok_r = token_emb.reshape(_VOCAB // 2, 2 * _EMBED)
    out6 = _run(xt_flat, tok_r, pos_emb)
    return out6.transpose(2, 0, 1)


# speedup vs baseline: 1.6679x; 1.1758x over previous
import functools

import jax
import jax.numpy as jnp
from jax import lax
from jax.experimental import pallas as pl
from jax.experimental.pallas import tpu as pltpu
from jax.experimental.pallas import tpu_sc as plsc

_VOCAB = 1000000
_MAXLEN = 200
_EMBED = 64
_BATCH = 1024

_INFO = plsc.get_sparse_core_info()
_NC, _NS, _L = _INFO.num_cores, _INFO.num_subcores, _INFO.num_lanes
_NW = _NC * _NS
_BB = _BATCH // 128
_UNITS = _MAXLEN * _BB
_UPW = _UNITS // _NW
_GPB = 128 // _L
_NB = 4
_ROUNDS = 12


def _body(xt_hbm, tok_hbm, pos_hbm, out_hbm,
          pos_v, ids_v, idx_v, par_v, bufs, slabs, isem, gsem, osem):
    wid = lax.axis_index("s") * _NC + lax.axis_index("c")
    ubase = wid * _UPW

    pltpu.sync_copy(pos_hbm, pos_v)

    def i_copy(ul, slot):
        return pltpu.make_async_copy(
            xt_hbm.at[pl.ds((ubase + ul) * 128, 128)],
            ids_v.at[slot], isem.at[slot])

    def prep(slot):
        for g in range(_GPB):
            sl = pl.ds(g * _L, _L)
            ids = ids_v[slot, sl]
            idx_v[slot, sl] = lax.shift_right_logical(ids, 1)
            par_v[slot, sl] = lax.shift_left(jnp.bitwise_and(ids, 1), 6)

    def g_copy(ul, slot):
        return pltpu.make_async_copy(
            tok_hbm.at[idx_v.at[slot]], bufs[slot], gsem.at[slot])

    def o_copy(ul, sslot):
        u = ubase + ul
        s = u >> 3
        bb = u & 7
        return pltpu.make_async_copy(
            slabs[sslot], out_hbm.at[s, :, pl.ds(bb * 128, 128)],
            osem.at[sslot])

    def transpose_add(ul, slot, sslot):
        u = ubase + ul
        s = u >> 3
        buf = bufs[slot]
        slab = slabs[sslot]
        zs = jnp.zeros((_L,), jnp.int32)
        iot = lax.iota(jnp.int32, _L)
        pars = [par_v[slot, pl.ds(g * _L, _L)] for g in range(_GPB)]

        @plsc.parallel_loop(0, _L, unroll=2)
        def _diag(d):
            rot = jnp.bitwise_and(iot + d, _L - 1)
            for E0 in range(0, _EMBED, _L):
                e_lanes = rot + E0
                p = plsc.load_gather(pos_v, [zs + s, e_lanes])
                for g in range(_GPB):
                    row_idx = iot + g * _L
                    col_idx = pars[g] + e_lanes
                    vals = plsc.load_gather(buf, [row_idx, col_idx])
                    plsc.store_scatter(slab, [e_lanes, row_idx], vals + p)

    for j in range(_NB):
        i_copy(j, j).start()
        i_copy(j, j).wait()
        prep(j)
        g_copy(j, j).start()
        i_copy(j + _NB, j).start()

    def step(i, j):
        ul = i * _NB + j
        g_copy(ul, j).wait()

        @pl.when(ul >= 2)
        def _():
            o_copy(ul - 2, j % 2).wait()

        transpose_add(ul, j, j % 2)
        o_copy(ul, j % 2).start()

        @pl.when(ul + _NB < _UPW)
        def _():
            i_copy(ul + _NB, j).wait()
            prep(j)
            g_copy(ul + _NB, j).start()

            @pl.when(ul + 2 * _NB < _UPW)
            def _():
                i_copy(ul + 2 * _NB, j).start()

    def round_body(i, carry):
        for j in range(_NB):
            step(i, j)
        return carry

    lax.fori_loop(0, _ROUNDS, round_body, 0)

    for j in range(2):
        step(_ROUNDS, j)

    o_copy(_UPW - 2, 0).wait()
    o_copy(_UPW - 1, 1).wait()


@jax.jit
def _run(xt_flat, tok_r, pos_e):
    mesh = plsc.VectorSubcoreMesh(core_axis_name="c", subcore_axis_name="s")

    def wrapped(xt_hbm, tok_hbm, pos_hbm, out_hbm, pos_v, ids_v, idx_v,
                par_v, b0, b1, b2, b3, s0, s1, isem, gsem, osem):
        _body(xt_hbm, tok_hbm, pos_hbm, out_hbm, pos_v, ids_v, idx_v,
              par_v, (b0, b1, b2, b3), (s0, s1), isem, gsem, osem)

    k = functools.partial(
        pl.kernel,
        mesh=mesh,
        out_type=jax.ShapeDtypeStruct((_MAXLEN, _EMBED, _BATCH), jnp.float32),
        scratch_types=[
            pltpu.VMEM((_MAXLEN, _EMBED), jnp.float32),
            pltpu.VMEM((_NB, 128), jnp.int32),
            pltpu.VMEM((_NB, 128), jnp.int32),
            pltpu.VMEM((_NB, 128), jnp.int32),
            pltpu.VMEM((128, 128), jnp.float32),
            pltpu.VMEM((128, 128), jnp.float32),
            pltpu.VMEM((128, 128), jnp.float32),
            pltpu.VMEM((128, 128), jnp.float32),
            pltpu.VMEM((_EMBED, 128), jnp.float32),
            pltpu.VMEM((_EMBED, 128), jnp.float32),
            pltpu.SemaphoreType.DMA((_NB,)),
            pltpu.SemaphoreType.DMA((_NB,)),
            pltpu.SemaphoreType.DMA((2,)),
        ],
        compiler_params=pltpu.CompilerParams(needs_layout_passes=False),
    )(wrapped)
    return k(xt_flat, tok_r, pos_e)


def kernel(x, token_emb, pos_emb):
    xt_flat = x.T.reshape(-1).astype(jnp.int32)
    tok_r = token_emb.reshape(_VOCAB // 2, 2 * _EMBED)
    out6 = _run(xt_flat, tok_r, pos_emb)
    return out6.transpose(2, 0, 1)


# speedup vs baseline: 3.5440x; 2.1248x over previous
import functools

import jax
import jax.numpy as jnp
from jax import lax
from jax.experimental import pallas as pl
from jax.experimental.pallas import tpu as pltpu
from jax.experimental.pallas import tpu_sc as plsc

_VOCAB = 1000000
_MAXLEN = 200
_EMBED = 64
_BATCH = 1024

_INFO = plsc.get_sparse_core_info()
_NC, _NS, _L = _INFO.num_cores, _INFO.num_subcores, _INFO.num_lanes
_NW = _NC * _NS
_GPB = 8
_NB = 4
_UPW = _MAXLEN * 8 // _NW
_ROUNDS = 12
_FT = _VOCAB // 128
_FROUNDS = (_FT // _NW + 1 + 1) // 2
_TAIL0 = _FT * 128
_PTAIL = _TAIL0 // 2


def _fmt_body(tokT_hbm, tail_hbm, outP_hbm, fbufs, pbufs, tb, fsem, psem):
    wid = lax.axis_index("s") * _NC + lax.axis_index("c")
    iot = lax.iota(jnp.int32, _L)

    def in_copy(c, slot):
        return pltpu.make_async_copy(
            tokT_hbm.at[:, pl.ds(c * 128, 128)], fbufs[slot], fsem.at[slot])

    def out_copy(c, slot):
        return pltpu.make_async_copy(
            pbufs[slot], outP_hbm.at[pl.ds(c * 64, 64)], psem.at[slot])

    def transpose(slot):
        fbuf = fbufs[slot]
        pbuf = pbufs[slot]

        @plsc.parallel_loop(0, _L, unroll=2)
        def _d(d):
            rot = jnp.bitwise_and(iot + d, _L - 1)
            for R0 in range(0, 64, _L):
                rr = rot + R0
                for g in range(_GPB):
                    row_idx = iot + 16 * (g & 3)
                    col_idx = 2 * rr + (g >> 2)
                    vals = plsc.load_gather(fbuf, [row_idx, col_idx])
                    plsc.store_scatter(pbuf, [rr, iot + 16 * g], vals)

    in_copy(wid, 0).start()
    in_copy(wid + _NW, 1).start()

    def round_body(i, carry):
        for slot in range(2):
            c = wid + (2 * i + slot) * _NW

            @pl.when(c < _FT)
            def _():
                in_copy(c, slot).wait()

                @pl.when(i >= 1)
                def _():
                    out_copy(c - 2 * _NW, slot).wait()

                transpose(slot)
                out_copy(c, slot).start()

                @pl.when(c + 2 * _NW < _FT)
                def _():
                    in_copy(c + 2 * _NW, slot).start()
        return carry

    lax.fori_loop(0, _FROUNDS, round_body, 0)

    out_copy(0, 0).wait()
    out_copy(0, 1).wait()

    @pl.when(wid == _NW - 1)
    def _():
        pltpu.sync_copy(tail_hbm, tb)

        @plsc.parallel_loop(0, 32)
        def _r(r):
            for g in range(_GPB):
                pbufs[0][r, pl.ds(16 * g, _L)] = (
                    tb[2 * r + (g >> 2), pl.ds(16 * (g & 3), _L)])

        pltpu.sync_copy(pbufs[0].at[pl.ds(0, 32), :],
                        outP_hbm.at[pl.ds(_PTAIL, 32)])


@jax.jit
def _fmt(tokT, tail):
    mesh = plsc.VectorSubcoreMesh(core_axis_name="c", subcore_axis_name="s")

    def wrapped(tokT_hbm, tail_hbm, outP_hbm, fb0, fb1, pb0, pb1, tb,
                fsem, psem):
        _fmt_body(tokT_hbm, tail_hbm, outP_hbm, (fb0, fb1), (pb0, pb1), tb,
                  fsem, psem)

    k = functools.partial(
        pl.kernel,
        mesh=mesh,
        out_type=jax.ShapeDtypeStruct((_VOCAB // 2, 128), jnp.float32),
        scratch_types=[
            pltpu.VMEM((_EMBED, 128), jnp.float32),
            pltpu.VMEM((_EMBED, 128), jnp.float32),
            pltpu.VMEM((_EMBED, 128), jnp.float32),
            pltpu.VMEM((_EMBED, 128), jnp.float32),
            pltpu.VMEM((_EMBED, _EMBED), jnp.float32),
            pltpu.SemaphoreType.DMA((2,)),
            pltpu.SemaphoreType.DMA((2,)),
        ],
        compiler_params=pltpu.CompilerParams(
            needs_layout_passes=False, use_tc_tiling_on_sc=True),
    )(wrapped)
    return k(tokT, tail)


def _run_body(xt_hbm, tok_hbm, pos_hbm, out_hbm,
              pos_v, ids_v, idx_v, par_v, bufs, slabs, isem, gsem, osem):
    wid = lax.axis_index("s") * _NC + lax.axis_index("c")
    ubase = wid * _UPW

    pltpu.sync_copy(pos_hbm, pos_v)

    def i_copy(ul, slot):
        return pltpu.make_async_copy(
            xt_hbm.at[pl.ds((ubase + ul) * 128, 128)],
            ids_v.at[slot], isem.at[slot])

    def prep(slot):
        for g in range(_GPB):
            sl = pl.ds(g * _L, _L)
            ids = ids_v[slot, sl]
            idx_v[slot, sl] = lax.shift_right_logical(ids, 1)
            par_v[slot, sl] = lax.shift_left(jnp.bitwise_and(ids, 1), 6)

    def g_copy(ul, slot):
        return pltpu.make_async_copy(
            tok_hbm.at[idx_v.at[slot]], bufs[slot], gsem.at[slot])

    def o_copy(ul, sslot):
        u = ubase + ul
        s = u >> 3
        bb = u & 7
        return pltpu.make_async_copy(
            slabs[sslot], out_hbm.at[s, :, pl.ds(bb * 128, 128)],
            osem.at[sslot])

    def transpose_add(ul, slot, sslot):
        u = ubase + ul
        s = u >> 3
        buf = bufs[slot]
        slab = slabs[sslot]
        zs = jnp.zeros((_L,), jnp.int32)
        iot = lax.iota(jnp.int32, _L)
        pars = [par_v[slot, pl.ds(g * _L, _L)] for g in range(_GPB)]

        @plsc.parallel_loop(0, _L, unroll=2)
        def _diag(d):
            rot = jnp.bitwise_and(iot + d, _L - 1)
            for E0 in range(0, _EMBED, _L):
                e_lanes = rot + E0
                p = plsc.load_gather(pos_v, [zs + s, e_lanes])
                for g in range(_GPB):
                    row_idx = iot + g * _L
                    col_idx = pars[g] + e_lanes
                    vals = plsc.load_gather(buf, [row_idx, col_idx])
                    plsc.store_scatter(slab, [e_lanes, row_idx], vals + p)

    for j in range(_NB):
        i_copy(j, j).start()
        i_copy(j, j).wait()
        prep(j)
        g_copy(j, j).start()
        i_copy(j + _NB, j).start()

    def step(i, j):
        ul = i * _NB + j
        g_copy(ul, j).wait()

        @pl.when(ul >= 2)
        def _():
            o_copy(ul - 2, j % 2).wait()

        transpose_add(ul, j, j % 2)
        o_copy(ul, j % 2).start()

        @pl.when(ul + _NB < _UPW)
        def _():
            i_copy(ul + _NB, j).wait()
            prep(j)
            g_copy(ul + _NB, j).start()

            @pl.when(ul + 2 * _NB < _UPW)
            def _():
                i_copy(ul + 2 * _NB, j).start()

    def round_body(i, carry):
        for j in range(_NB):
            step(i, j)
        return carry

    lax.fori_loop(0, _ROUNDS, round_body, 0)

    for j in range(2):
        step(_ROUNDS, j)

    o_copy(_UPW - 2, 0).wait()
    o_copy(_UPW - 1, 1).wait()


@jax.jit
def _run(xt_flat, tok_r, pos_e):
    mesh = plsc.VectorSubcoreMesh(core_axis_name="c", subcore_axis_name="s")

    def wrapped(xt_hbm, tok_hbm, pos_hbm, out_hbm, pos_v, ids_v, idx_v,
                par_v, b0, b1, b2, b3, s0, s1, isem, gsem, osem):
        _run_body(xt_hbm, tok_hbm, pos_hbm, out_hbm, pos_v, ids_v, idx_v,
                  par_v, (b0, b1, b2, b3), (s0, s1), isem, gsem, osem)

    k = functools.partial(
        pl.kernel,
        mesh=mesh,
        out_type=jax.ShapeDtypeStruct((_MAXLEN, _EMBED, _BATCH), jnp.float32),
        scratch_types=[
            pltpu.VMEM((_MAXLEN, _EMBED), jnp.float32),
            pltpu.VMEM((_NB, 128), jnp.int32),
            pltpu.VMEM((_NB, 128), jnp.int32),
            pltpu.VMEM((_NB, 128), jnp.int32),
            pltpu.VMEM((128, 128), jnp.float32),
            pltpu.VMEM((128, 128), jnp.float32),
            pltpu.VMEM((128, 128), jnp.float32),
            pltpu.VMEM((128, 128), jnp.float32),
            pltpu.VMEM((_EMBED, 128), jnp.float32),
            pltpu.VMEM((_EMBED, 128), jnp.float32),
            pltpu.SemaphoreType.DMA((_NB,)),
            pltpu.SemaphoreType.DMA((_NB,)),
            pltpu.SemaphoreType.DMA((2,)),
        ],
        compiler_params=pltpu.CompilerParams(
            needs_layout_passes=False, use_tc_tiling_on_sc=True),
    )(wrapped)
    return k(xt_flat, tok_r, pos_e)


def kernel(x, token_emb, pos_emb):
    xt_flat = x.T.reshape(-1).astype(jnp.int32)
    tok_r = _fmt(token_emb.T, token_emb[_TAIL0:, :])
    out6 = _run(xt_flat, tok_r, pos_emb)
    return out6.transpose(2, 0, 1)


# speedup vs baseline: 3.6173x; 1.0207x over previous
import functools

import jax
import jax.numpy as jnp
from jax import lax
from jax.experimental import pallas as pl
from jax.experimental.pallas import tpu as pltpu
from jax.experimental.pallas import tpu_sc as plsc

_VOCAB = 1000000
_MAXLEN = 200
_EMBED = 64
_BATCH = 1024

_INFO = plsc.get_sparse_core_info()
_NC, _NS, _L = _INFO.num_cores, _INFO.num_subcores, _INFO.num_lanes
_NW = _NC * _NS
_GPB = 8
_NB = 4
_UPW = _MAXLEN * 8 // _NW
_ROUNDS = 12
_FT = _VOCAB // 128
_FROUNDS = (_FT // _NW + 2 + 3) // 4
_TAIL0 = _FT * 128
_PTAIL = _TAIL0 // 2


def _fmt_body(tokT_hbm, tail_hbm, outP_hbm, fbufs, pbufs, tb, fsem, psem):
    wid = lax.axis_index("s") * _NC + lax.axis_index("c")
    iot = lax.iota(jnp.int32, _L)

    def in_copy(c, slot):
        return pltpu.make_async_copy(
            tokT_hbm.at[:, pl.ds(c * 128, 128)], fbufs[slot], fsem.at[slot])

    def out_copy(c, slot):
        return pltpu.make_async_copy(
            pbufs[slot], outP_hbm.at[pl.ds(c * 64, 64)], psem.at[slot])

    def transpose(slot):
        fbuf = fbufs[slot]
        pbuf = pbufs[slot]

        @plsc.parallel_loop(0, _L, unroll=4)
        def _d(d):
            rot = jnp.bitwise_and(iot + d, _L - 1)
            for R0 in range(0, 64, _L):
                rr = rot + R0
                for g in range(_GPB):
                    row_idx = iot + 16 * (g & 3)
                    col_idx = 2 * rr + (g >> 2)
                    vals = plsc.load_gather(fbuf, [row_idx, col_idx])
                    plsc.store_scatter(pbuf, [rr, iot + 16 * g], vals)

    for j in range(4):
        in_copy(wid + j * _NW, j).start()

    def round_body(i, carry):
        for slot in range(4):
            c = wid + (4 * i + slot) * _NW

            @pl.when(c < _FT)
            def _():
                in_copy(c, slot).wait()

                @pl.when(i >= 1)
                def _():
                    out_copy(c - 4 * _NW, slot).wait()

                transpose(slot)
                out_copy(c, slot).start()

                @pl.when(c + 4 * _NW < _FT)
                def _():
                    in_copy(c + 4 * _NW, slot).start()
        return carry

    lax.fori_loop(0, _FROUNDS, round_body, 0)

    for j in range(4):
        out_copy(0, j).wait()

    @pl.when(wid == _NW - 1)
    def _():
        pltpu.sync_copy(tail_hbm, tb)

        @plsc.parallel_loop(0, 32)
        def _r(r):
            for g in range(_GPB):
                pbufs[0][r, pl.ds(16 * g, _L)] = (
                    tb[2 * r + (g >> 2), pl.ds(16 * (g & 3), _L)])

        pltpu.sync_copy(pbufs[0].at[pl.ds(0, 32), :],
                        outP_hbm.at[pl.ds(_PTAIL, 32)])


@jax.jit
def _fmt(tokT, tail):
    mesh = plsc.VectorSubcoreMesh(core_axis_name="c", subcore_axis_name="s")

    def wrapped(tokT_hbm, tail_hbm, outP_hbm, fb0, fb1, fb2, fb3,
                pb0, pb1, pb2, pb3, tb, fsem, psem):
        _fmt_body(tokT_hbm, tail_hbm, outP_hbm, (fb0, fb1, fb2, fb3),
                  (pb0, pb1, pb2, pb3), tb, fsem, psem)

    k = functools.partial(
        pl.kernel,
        mesh=mesh,
        out_type=jax.ShapeDtypeStruct((_VOCAB // 2, 128), jnp.float32),
        scratch_types=[
            pltpu.VMEM((_EMBED, 128), jnp.float32),
            pltpu.VMEM((_EMBED, 128), jnp.float32),
            pltpu.VMEM((_EMBED, 128), jnp.float32),
            pltpu.VMEM((_EMBED, 128), jnp.float32),
            pltpu.VMEM((_EMBED, 128), jnp.float32),
            pltpu.VMEM((_EMBED, 128), jnp.float32),
            pltpu.VMEM((_EMBED, 128), jnp.float32),
            pltpu.VMEM((_EMBED, 128), jnp.float32),
            pltpu.VMEM((_EMBED, _EMBED), jnp.float32),
            pltpu.SemaphoreType.DMA((4,)),
            pltpu.SemaphoreType.DMA((4,)),
        ],
        compiler_params=pltpu.CompilerParams(
            needs_layout_passes=False, use_tc_tiling_on_sc=True),
    )(wrapped)
    return k(tokT, tail)


def _run_body(xt_hbm, tok_hbm, pos_hbm, out_hbm,
              pos_v, ids_v, idx_v, par_v, bufs, slabs, isem, gsem, osem):
    wid = lax.axis_index("s") * _NC + lax.axis_index("c")
    ubase = wid * _UPW

    pltpu.sync_copy(pos_hbm, pos_v)

    def i_copy(ul, slot):
        return pltpu.make_async_copy(
            xt_hbm.at[pl.ds((ubase + ul) * 128, 128)],
            ids_v.at[slot], isem.at[slot])

    def prep(slot):
        for g in range(_GPB):
            sl = pl.ds(g * _L, _L)
            ids = ids_v[slot, sl]
            idx_v[slot, sl] = lax.shift_right_logical(ids, 1)
            par_v[slot, sl] = lax.shift_left(jnp.bitwise_and(ids, 1), 6)

    def g_copy(ul, slot):
        return pltpu.make_async_copy(
            tok_hbm.at[idx_v.at[slot]], bufs[slot], gsem.at[slot])

    def o_copy(ul, sslot):
        u = ubase + ul
        s = u >> 3
        bb = u & 7
        return pltpu.make_async_copy(
            slabs[sslot], out_hbm.at[s, :, pl.ds(bb * 128, 128)],
            osem.at[sslot])

    def transpose_add(ul, slot, sslot):
        u = ubase + ul
        s = u >> 3
        buf = bufs[slot]
        slab = slabs[sslot]
        zs = jnp.zeros((_L,), jnp.int32)
        iot = lax.iota(jnp.int32, _L)
        pars = [par_v[slot, pl.ds(g * _L, _L)] for g in range(_GPB)]

        @plsc.parallel_loop(0, _L, unroll=2)
        def _diag(d):
            rot = jnp.bitwise_and(iot + d, _L - 1)
            for E0 in range(0, _EMBED, _L):
                e_lanes = rot + E0
                p = plsc.load_gather(pos_v, [zs + s, e_lanes])
                for g in range(_GPB):
                    row_idx = iot + g * _L
                    col_idx = pars[g] + e_lanes
                    vals = plsc.load_gather(buf, [row_idx, col_idx])
                    plsc.store_scatter(slab, [e_lanes, row_idx], vals + p)

    for j in range(_NB):
        i_copy(j, j).start()
        i_copy(j, j).wait()
        prep(j)
        g_copy(j, j).start()
        i_copy(j + _NB, j).start()

    def step(i, j):
        ul = i * _NB + j
        g_copy(ul, j).wait()

        @pl.when(ul >= 2)
        def _():
            o_copy(ul - 2, j % 2).wait()

        transpose_add(ul, j, j % 2)
        o_copy(ul, j % 2).start()

        @pl.when(ul + _NB < _UPW)
        def _():
            i_copy(ul + _NB, j).wait()
            prep(j)
            g_copy(ul + _NB, j).start()

            @pl.when(ul + 2 * _NB < _UPW)
            def _():
                i_copy(ul + 2 * _NB, j).start()

    def round_body(i, carry):
        for j in range(_NB):
            step(i, j)
        return carry

    lax.fori_loop(0, _ROUNDS, round_body, 0)

    for j in range(2):
        step(_ROUNDS, j)

    o_copy(_UPW - 2, 0).wait()
    o_copy(_UPW - 1, 1).wait()


@jax.jit
def _run(xt_flat, tok_r, pos_e):
    mesh = plsc.VectorSubcoreMesh(core_axis_name="c", subcore_axis_name="s")

    def wrapped(xt_hbm, tok_hbm, pos_hbm, out_hbm, pos_v, ids_v, idx_v,
                par_v, b0, b1, b2, b3, s0, s1, isem, gsem, osem):
        _run_body(xt_hbm, tok_hbm, pos_hbm, out_hbm, pos_v, ids_v, idx_v,
                  par_v, (b0, b1, b2, b3), (s0, s1), isem, gsem, osem)

    k = functools.partial(
        pl.kernel,
        mesh=mesh,
        out_type=jax.ShapeDtypeStruct((_MAXLEN, _EMBED, _BATCH), jnp.float32),
        scratch_types=[
            pltpu.VMEM((_MAXLEN, _EMBED), jnp.float32),
            pltpu.VMEM((_NB, 128), jnp.int32),
            pltpu.VMEM((_NB, 128), jnp.int32),
            pltpu.VMEM((_NB, 128), jnp.int32),
            pltpu.VMEM((128, 128), jnp.float32),
            pltpu.VMEM((128, 128), jnp.float32),
            pltpu.VMEM((128, 128), jnp.float32),
            pltpu.VMEM((128, 128), jnp.float32),
            pltpu.VMEM((_EMBED, 128), jnp.float32),
            pltpu.VMEM((_EMBED, 128), jnp.float32),
            pltpu.SemaphoreType.DMA((_NB,)),
            pltpu.SemaphoreType.DMA((_NB,)),
            pltpu.SemaphoreType.DMA((2,)),
        ],
        compiler_params=pltpu.CompilerParams(
            needs_layout_passes=False, use_tc_tiling_on_sc=True),
    )(wrapped)
    return k(xt_flat, tok_r, pos_e)


def kernel(x, token_emb, pos_emb):
    xt_flat = x.T.reshape(-1).astype(jnp.int32)
    tok_r = _fmt(token_emb.T, token_emb[_TAIL0:, :])
    out6 = _run(xt_flat, tok_r, pos_emb)
    return out6.transpose(2, 0, 1)


# speedup vs baseline: 3.7609x; 1.0397x over previous
import functools

import jax
import jax.numpy as jnp
from jax import lax
from jax.experimental import pallas as pl
from jax.experimental.pallas import tpu as pltpu
from jax.experimental.pallas import tpu_sc as plsc

_VOCAB = 1000000
_MAXLEN = 200
_EMBED = 64
_BATCH = 1024

_INFO = plsc.get_sparse_core_info()
_NC, _NS, _L = _INFO.num_cores, _INFO.num_subcores, _INFO.num_lanes
_NW = _NC * _NS
_GPB = 8
_NB = 4
_UPW = _MAXLEN * 8 // _NW
_ROUNDS = 12
_FT = _VOCAB // 128
_FROUNDS = (_FT // _NW + 2 + 3) // 4
_TAIL0 = _FT * 128
_PTAIL = _TAIL0 // 2


def _fmt_body(tokT_hbm, tail_hbm, outP_hbm, fbufs, pbufs, tb, fsem, psem):
    wid = lax.axis_index("s") * _NC + lax.axis_index("c")
    iot = lax.iota(jnp.int32, _L)

    def in_copy(c, slot):
        return pltpu.make_async_copy(
            tokT_hbm.at[:, pl.ds(c * 128, 128)], fbufs[slot], fsem.at[slot])

    def out_copy(c, slot):
        return pltpu.make_async_copy(
            pbufs[slot], outP_hbm.at[pl.ds(c * 64, 64)], psem.at[slot])

    def transpose(slot):
        fbuf = fbufs[slot]
        pbuf = pbufs[slot]

        @plsc.parallel_loop(0, _L, unroll=8)
        def _d(d):
            rot = jnp.bitwise_and(iot + d, _L - 1)
            for R0 in range(0, 64, _L):
                rr = rot + R0
                for g in range(_GPB):
                    row_idx = iot + 16 * (g & 3)
                    col_idx = 2 * rr + (g >> 2)
                    vals = plsc.load_gather(fbuf, [row_idx, col_idx])
                    plsc.store_scatter(pbuf, [rr, iot + 16 * g], vals)

    for j in range(4):
        in_copy(wid + j * _NW, j).start()

    def round_body(i, carry):
        for slot in range(4):
            c = wid + (4 * i + slot) * _NW

            @pl.when(c < _FT)
            def _():
                in_copy(c, slot).wait()

                @pl.when(i >= 1)
                def _():
                    out_copy(c - 4 * _NW, slot).wait()

                transpose(slot)
                out_copy(c, slot).start()

                @pl.when(c + 4 * _NW < _FT)
                def _():
                    in_copy(c + 4 * _NW, slot).start()
        return carry

    lax.fori_loop(0, _FROUNDS, round_body, 0)

    for j in range(4):
        out_copy(0, j).wait()

    @pl.when(wid == _NW - 1)
    def _():
        pltpu.sync_copy(tail_hbm, tb)

        @plsc.parallel_loop(0, 32)
        def _r(r):
            for g in range(_GPB):
                pbufs[0][r, pl.ds(16 * g, _L)] = (
                    tb[2 * r + (g >> 2), pl.ds(16 * (g & 3), _L)])

        pltpu.sync_copy(pbufs[0].at[pl.ds(0, 32), :],
                        outP_hbm.at[pl.ds(_PTAIL, 32)])


@jax.jit
def _fmt(tokT, tail):
    mesh = plsc.VectorSubcoreMesh(core_axis_name="c", subcore_axis_name="s")

    def wrapped(tokT_hbm, tail_hbm, outP_hbm, fb0, fb1, fb2, fb3,
                pb0, pb1, pb2, pb3, tb, fsem, psem):
        _fmt_body(tokT_hbm, tail_hbm, outP_hbm, (fb0, fb1, fb2, fb3),
                  (pb0, pb1, pb2, pb3), tb, fsem, psem)

    k = functools.partial(
        pl.kernel,
        mesh=mesh,
        out_type=jax.ShapeDtypeStruct((_VOCAB // 2, 128), jnp.float32),
        scratch_types=[
            pltpu.VMEM((_EMBED, 128), jnp.float32),
            pltpu.VMEM((_EMBED, 128), jnp.float32),
            pltpu.VMEM((_EMBED, 128), jnp.float32),
            pltpu.VMEM((_EMBED, 128), jnp.float32),
            pltpu.VMEM((_EMBED, 128), jnp.float32),
            pltpu.VMEM((_EMBED, 128), jnp.float32),
            pltpu.VMEM((_EMBED, 128), jnp.float32),
            pltpu.VMEM((_EMBED, 128), jnp.float32),
            pltpu.VMEM((_EMBED, _EMBED), jnp.float32),
            pltpu.SemaphoreType.DMA((4,)),
            pltpu.SemaphoreType.DMA((4,)),
        ],
        compiler_params=pltpu.CompilerParams(
            needs_layout_passes=False, use_tc_tiling_on_sc=True),
    )(wrapped)
    return k(tokT, tail)


def _run_body(xt_hbm, tok_hbm, pos_hbm, out_hbm,
              pos_v, ids_v, idx_v, par_v, bufs, slabs, isem, gsem, osem):
    wid = lax.axis_index("s") * _NC + lax.axis_index("c")
    ubase = wid * _UPW

    pltpu.sync_copy(pos_hbm, pos_v)

    def i_copy(ul, slot):
        return pltpu.make_async_copy(
            xt_hbm.at[pl.ds((ubase + ul) * 128, 128)],
            ids_v.at[slot], isem.at[slot])

    def prep(slot):
        for g in range(_GPB):
            sl = pl.ds(g * _L, _L)
            ids = ids_v[slot, sl]
            idx_v[slot, sl] = lax.shift_right_logical(ids, 1)
            par_v[slot, sl] = lax.shift_left(jnp.bitwise_and(ids, 1), 6)

    def g_copy(ul, slot):
        return pltpu.make_async_copy(
            tok_hbm.at[idx_v.at[slot]], bufs[slot], gsem.at[slot])

    def o_copy(ul, sslot):
        u = ubase + ul
        s = u >> 3
        bb = u & 7
        return pltpu.make_async_copy(
            slabs[sslot], out_hbm.at[s, :, pl.ds(bb * 128, 128)],
            osem.at[sslot])

    def transpose_add(ul, slot, sslot):
        u = ubase + ul
        s = u >> 3
        buf = bufs[slot]
        slab = slabs[sslot]
        zs = jnp.zeros((_L,), jnp.int32)
        iot = lax.iota(jnp.int32, _L)
        pars = [par_v[slot, pl.ds(g * _L, _L)] for g in range(_GPB)]

        @plsc.parallel_loop(0, _L, unroll=4)
        def _diag(d):
            rot = jnp.bitwise_and(iot + d, _L - 1)
            for E0 in range(0, _EMBED, _L):
                e_lanes = rot + E0
                p = plsc.load_gather(pos_v, [zs + s, e_lanes])
                for g in range(_GPB):
                    row_idx = iot + g * _L
                    col_idx = pars[g] + e_lanes
                    vals = plsc.load_gather(buf, [row_idx, col_idx])
                    plsc.store_scatter(slab, [e_lanes, row_idx], vals + p)

    for j in range(_NB):
        i_copy(j, j).start()
        i_copy(j, j).wait()
        prep(j)
        g_copy(j, j).start()
        i_copy(j + _NB, j).start()

    def step(i, j):
        ul = i * _NB + j
        g_copy(ul, j).wait()

        @pl.when(ul >= 2)
        def _():
            o_copy(ul - 2, j % 2).wait()

        transpose_add(ul, j, j % 2)
        o_copy(ul, j % 2).start()

        @pl.when(ul + _NB < _UPW)
        def _():
            i_copy(ul + _NB, j).wait()
            prep(j)
            g_copy(ul + _NB, j).start()

            @pl.when(ul + 2 * _NB < _UPW)
            def _():
                i_copy(ul + 2 * _NB, j).start()

    def round_body(i, carry):
        for j in range(_NB):
            step(i, j)
        return carry

    lax.fori_loop(0, _ROUNDS, round_body, 0)

    for j in range(2):
        step(_ROUNDS, j)

    o_copy(_UPW - 2, 0).wait()
    o_copy(_UPW - 1, 1).wait()


@jax.jit
def _run(xt_flat, tok_r, pos_e):
    mesh = plsc.VectorSubcoreMesh(core_axis_name="c", subcore_axis_name="s")

    def wrapped(xt_hbm, tok_hbm, pos_hbm, out_hbm, pos_v, ids_v, idx_v,
                par_v, b0, b1, b2, b3, s0, s1, isem, gsem, osem):
        _run_body(xt_hbm, tok_hbm, pos_hbm, out_hbm, pos_v, ids_v, idx_v,
                  par_v, (b0, b1, b2, b3), (s0, s1), isem, gsem, osem)

    k = functools.partial(
        pl.kernel,
        mesh=mesh,
        out_type=jax.ShapeDtypeStruct((_MAXLEN, _EMBED, _BATCH), jnp.float32),
        scratch_types=[
            pltpu.VMEM((_MAXLEN, _EMBED), jnp.float32),
            pltpu.VMEM((_NB, 128), jnp.int32),
            pltpu.VMEM((_NB, 128), jnp.int32),
            pltpu.VMEM((_NB, 128), jnp.int32),
            pltpu.VMEM((128, 128), jnp.float32),
            pltpu.VMEM((128, 128), jnp.float32),
            pltpu.VMEM((128, 128), jnp.float32),
            pltpu.VMEM((128, 128), jnp.float32),
            pltpu.VMEM((_EMBED, 128), jnp.float32),
            pltpu.VMEM((_EMBED, 128), jnp.float32),
            pltpu.SemaphoreType.DMA((_NB,)),
            pltpu.SemaphoreType.DMA((_NB,)),
            pltpu.SemaphoreType.DMA((2,)),
        ],
        compiler_params=pltpu.CompilerParams(
            needs_layout_passes=False, use_tc_tiling_on_sc=True),
    )(wrapped)
    return k(xt_flat, tok_r, pos_e)


def kernel(x, token_emb, pos_emb):
    xt_flat = x.T.reshape(-1).astype(jnp.int32)
    tok_r = _fmt(token_emb.T, token_emb[_TAIL0:, :])
    out6 = _run(xt_flat, tok_r, pos_emb)
    return out6.transpose(2, 0, 1)
